# Initial kernel scaffold; baseline (speedup 1.0000x reference)
import jax
import jax.numpy as jnp
from jax.experimental import pallas as pl


def kernel(drug1_id, drug2_id, cell_features, x_drug, x_target, edge_dd, edge_dt, edge_rdt, edge_tt, drug_mask, target_mask, params):
    raise NotImplementedError("write your pallas kernel here")


# speedup vs baseline: 1.6703x; 1.6703x over previous
import functools

import jax
import jax.numpy as jnp
from jax.experimental import pallas as pl
from jax.experimental.pallas import tpu as pltpu

Nd = 10000
Nt = 10000
H = 768


def _rup(x, m):
    return (x + m - 1) // m * m


def _mm_body(x_ref, w_ref, b_ref, o_ref, *, act, l2norm):
    x = x_ref[...]
    if l2norm:
        n = jnp.sqrt(jnp.sum(x * x, axis=1, keepdims=True))
        x = x / jnp.maximum(n, 1e-12)
    y = jnp.dot(x, w_ref[...], preferred_element_type=jnp.float32)
    y = y + b_ref[...]
    if act == "relu":
        y = jnp.maximum(y, 0.0)
    o_ref[...] = y


def _mm(x, w, b=None, act=None, l2norm=False, block_m=512):
    M, K = x.shape
    _, N = w.shape
    Mp, Kp, Np = _rup(M, block_m), _rup(K, 128), _rup(N, 128)
    xp = jnp.pad(x, ((0, Mp - M), (0, Kp - K)))
    wp = jnp.pad(w, ((0, Kp - K), (0, Np - N)))
    if b is None:
        bp = jnp.zeros((1, Np), jnp.float32)
    else:
        bp = jnp.pad(b.reshape(1, N), ((0, 0), (0, Np - N)))
    out = pl.pallas_call(
        functools.partial(_mm_body, act=act, l2norm=l2norm),
        grid=(Mp // block_m,),
        in_specs=[
            pl.BlockSpec((block_m, Kp), lambda i: (i, 0)),
            pl.BlockSpec((Kp, Np), lambda i: (0, 0)),
            pl.BlockSpec((1, Np), lambda i: (0, 0)),
        ],
        out_specs=pl.BlockSpec((block_m, Np), lambda i: (i, 0)),
        out_shape=jax.ShapeDtypeStruct((Mp, Np), jnp.float32),
    )(xp, wp, bp)
    return out[:M, :N]


def _gat_edges(hs, a_s, a_d, src, dst, num_dst):
    alpha = a_s[src] + a_d[dst]
    alpha = jnp.where(alpha > 0, alpha, 0.2 * alpha)
    A = jnp.maximum(jnp.max(a_s), 0.0) + jnp.maximum(jnp.max(a_d), 0.0)
    ex = jnp.exp(alpha - A)
    denom = jax.ops.segment_sum(ex, dst, num_segments=num_dst)
    acc = jax.ops.segment_sum(ex[:, None] * hs[src], dst, num_segments=num_dst)
    return acc, denom


def _finish(acc_a, den_a, b_a, acc_b, den_b, b_b):
    out = (acc_a / (den_a[:, None] + 1e-16) + b_a) + (acc_b / (den_b[:, None] + 1e-16) + b_b)
    return jnp.maximum(out, 0.0)


def _cos_loss(a, b):
    na = jnp.sqrt(jnp.sum(a * a, axis=1))
    nb = jnp.sqrt(jnp.sum(b * b, axis=1))
    cos = jnp.sum(a * b, axis=1) / jnp.maximum(na * nb, 1e-8)
    return jnp.sum(1.0 - cos)


def kernel(drug1_id, drug2_id, cell_features, x_drug, x_target, edge_dd, edge_dt, edge_rdt, edge_tt, drug_mask, target_mask, params):
    p = params
    _xd = x_drug * (1.0 - drug_mask) + drug_mask * p["mask_drug"]
    _xt = x_target * (1.0 - target_mask) + target_mask * p["mask_target"]

    loops_d = jnp.arange(Nd, dtype=jnp.int32)
    dd_src = jnp.concatenate([edge_dd[0], loops_d])
    dd_dst = jnp.concatenate([edge_dd[1], loops_d])
    loops_t = jnp.arange(Nt, dtype=jnp.int32)
    tt_src = jnp.concatenate([edge_tt[0], loops_t])
    tt_dst = jnp.concatenate([edge_tt[1], loops_t])

    vd = jnp.stack(
        [
            p["dd"]["Ws"] @ p["dd"]["att_s"],
            p["dd"]["Wd"] @ p["dd"]["att_d"],
            p["dt"]["Ws"] @ p["dt"]["att_s"],
            p["rdt"]["Wd"] @ p["rdt"]["att_d"],
        ],
        axis=1,
    )
    wd_cat = jnp.concatenate([p["dd"]["Ws"], p["dt"]["Ws"], vd], axis=1)
    vt = jnp.stack(
        [
            p["rdt"]["Ws"] @ p["rdt"]["att_s"],
            p["dt"]["Wd"] @ p["dt"]["att_d"],
            p["tt"]["Ws"] @ p["tt"]["att_s"],
            p["tt"]["Wd"] @ p["tt"]["att_d"],
        ],
        axis=1,
    )
    wt_cat = jnp.concatenate([p["rdt"]["Ws"], p["tt"]["Ws"], vt], axis=1)

    xd2 = jnp.concatenate([_xd, x_drug], axis=0)
    xt2 = jnp.concatenate([_xt, x_target], axis=0)
    hd_cat = _mm(xd2, wd_cat)
    ht_cat = _mm(xt2, wt_cat)

    outs = {}
    for li, off in ((0, 0), (1, Nd)):
        hs_dd = hd_cat[off : off + Nd, 0:H]
        hs_dt = hd_cat[off : off + Nd, H : 2 * H]
        as_dd = hd_cat[off : off + Nd, 2 * H + 0]
        ad_dd = hd_cat[off : off + Nd, 2 * H + 1]
        as_dt = hd_cat[off : off + Nd, 2 * H + 2]
        ad_rdt = hd_cat[off : off + Nd, 2 * H + 3]
        hs_rdt = ht_cat[off : off + Nt, 0:H]
        hs_tt = ht_cat[off : off + Nt, H : 2 * H]
        as_rdt = ht_cat[off : off + Nt, 2 * H + 0]
        ad_dt = ht_cat[off : off + Nt, 2 * H + 1]
        as_tt = ht_cat[off : off + Nt, 2 * H + 2]
        ad_tt = ht_cat[off : off + Nt, 2 * H + 3]

        acc_dd, den_dd = _gat_edges(hs_dd, as_dd, ad_dd, dd_src, dd_dst, Nd)
        acc_rdt, den_rdt = _gat_edges(hs_rdt, as_rdt, ad_rdt, edge_rdt[0], edge_rdt[1], Nd)
        acc_dt, den_dt = _gat_edges(hs_dt, as_dt, ad_dt, edge_dt[0], edge_dt[1], Nt)
        acc_tt, den_tt = _gat_edges(hs_tt, as_tt, ad_tt, tt_src, tt_dst, Nt)

        outs[("d", li)] = _finish(acc_dd, den_dd, p["dd"]["b"], acc_rdt, den_rdt, p["rdt"]["b"])
        outs[("t", li)] = _finish(acc_dt, den_dt, p["dt"]["b"], acc_tt, den_tt, p["tt"]["b"])

    _xd_o, xd_o = outs[("d", 0)], outs[("d", 1)]
    _xt_o, xt_o = outs[("t", 0)], outs[("t", 1)]

    d_loss = _cos_loss(_xd_o * drug_mask, xd_o * drug_mask) / Nd
    t_loss = _cos_loss(_xt_o * target_mask, xt_o * target_mask) / Nt
    loss_mae = d_loss + t_loss

    drug1 = xd_o[drug1_id]
    drug2 = xd_o[drug2_id]

    cell = cell_features
    r0w, r0b = p["red"][0]
    cell = _mm(cell, r0w, r0b, act="relu", l2norm=True)
    r1w, r1b = p["red"][1]
    cell = _mm(cell, r1w, r1b, act="relu")
    r2w, r2b = p["red"][2]
    cell = _mm(cell, r2w, r2b, act="relu")

    hidden = jnp.concatenate([drug1, drug2, cell], axis=1)
    q0w, q0b = p["red2"][0]
    hidden = _mm(hidden, q0w, q0b, act="relu", l2norm=True)
    q1w, q1b = p["red2"][1]
    hidden = _mm(hidden, q1w, q1b, act="relu")
    q2w, q2b = p["red2"][2]
    hidden = _mm(hidden, q2w, q2b, act="relu")

    cw, cb = p["cls"]
    output = _mm(hidden, cw, cb)
    return output, loss_mae


# speedup vs baseline: 2.6975x; 1.6150x over previous
import functools

import jax
import jax.numpy as jnp
from jax import lax
from jax.experimental import pallas as pl
from jax.experimental.pallas import tpu as pltpu
from jax.experimental.pallas import tpu_sc as plsc

Nd = 10000
Nt = 10000
H = 768
NP = 10240
NSL = 6
EPS_DEN = 1e-16


def _rup(x, m):
    return (x + m - 1) // m * m


def _mm_body(x_ref, w_ref, b_ref, o_ref, *, act, l2norm):
    x = x_ref[...]
    if l2norm:
        n = jnp.sqrt(jnp.sum(x * x, axis=1, keepdims=True))
        x = x / jnp.maximum(n, 1e-12)
    y = jnp.dot(x, w_ref[...], preferred_element_type=jnp.float32)
    y = y + b_ref[...]
    if act == "relu":
        y = jnp.maximum(y, 0.0)
    o_ref[...] = y


def _mm(x, w, b=None, act=None, l2norm=False, block_m=512):
    M, K = x.shape
    _, N = w.shape
    Mp, Kp, Np = _rup(M, block_m), _rup(K, 128), _rup(N, 128)
    xp = jnp.pad(x, ((0, Mp - M), (0, Kp - K)))
    wp = jnp.pad(w, ((0, Kp - K), (0, Np - N)))
    if b is None:
        bp = jnp.zeros((1, Np), jnp.float32)
    else:
        bp = jnp.pad(b.reshape(1, N), ((0, 0), (0, Np - N)))
    out = pl.pallas_call(
        functools.partial(_mm_body, act=act, l2norm=l2norm),
        grid=(Mp // block_m,),
        in_specs=[
            pl.BlockSpec((block_m, Kp), lambda i: (i, 0)),
            pl.BlockSpec((Kp, Np), lambda i: (0, 0)),
            pl.BlockSpec((1, Np), lambda i: (0, 0)),
        ],
        out_specs=pl.BlockSpec((block_m, Np), lambda i: (i, 0)),
        out_shape=jax.ShapeDtypeStruct((Mp, Np), jnp.float32),
    )(xp, wp, bp)
    if Mp == M and Np == N:
        return out
    return out[:M, :N]


def _edge_kernel_body(
    nbs,
    srcA, dstA, asA, adA, cA, hsA,
    srcB, dstB, asB, adB, cB, hsB,
    accA, accB, den,
    sblk, dblk, asg, adg, exb, dstb, idxb, c16, rowv, zbuf, fbuf, zden, dbuf,
    accS, sdenA, sdenB, semg,
):
    nbA, nbB = nbs
    core = lax.axis_index("c")
    tid = lax.axis_index("s")

    def _z64(r, _):
        for j in range(8):
            zbuf[r, pl.ds(16 * j, 16)] = jnp.zeros((16,), jnp.float32)
        return 0

    lax.fori_loop(0, 64, _z64, 0)

    def _z640(i, _):
        zden[pl.ds(16 * i, 16)] = jnp.zeros((16,), jnp.float32)
        return 0

    lax.fori_loop(0, 40, _z640, 0)

    rels = (
        (nbA, srcA, dstA, asA, adA, cA, hsA, accA, sdenA, 0),
        (nbB, srcB, dstB, asB, adB, cB, hsB, accB, sdenB, 1),
    )

    def _load_ex(q, nb, src_h, dst_h, as_h, ad_h):
        base = (tid * nb + q) * 1024
        pltpu.sync_copy(src_h.at[pl.ds(base, 1024)], sblk)
        pltpu.sync_copy(dst_h.at[pl.ds(base, 1024)], dblk)
        for j in range(8):
            pltpu.async_copy(
                as_h.at[sblk.at[pl.ds(128 * j, 128)]], asg.at[pl.ds(128 * j, 128)], semg
            ).wait()
            pltpu.async_copy(
                ad_h.at[dblk.at[pl.ds(128 * j, 128)]], adg.at[pl.ds(128 * j, 128)], semg
            ).wait()
        for j in range(8):
            for i in range(8):
                o = 128 * j + 16 * i
                a = asg[pl.ds(o, 16)] + adg[pl.ds(o, 16)]
                a = jnp.where(a > 0, a, 0.2 * a)
                exb[j, pl.ds(16 * i, 16)] = jnp.exp(a - c16[...])
                dstb[j, pl.ds(16 * i, 16)] = dblk[pl.ds(o, 16)]

    for nb, src_h, dst_h, as_h, ad_h, c_h, hs_h, acc_h, sden, ridx in rels:
        pltpu.sync_copy(c_h, c16)

        pltpu.sync_copy(zden, sden.at[pl.ds(tid * 640, 640)])
        plsc.subcore_barrier()

        def _den_block(q, _):
            _load_ex(q, nb, src_h, dst_h, as_h, ad_h)
            for j in range(8):
                pltpu.sync_copy(exb.at[j], sden.at[dstb.at[j]], add=True)
            return 0

        lax.fori_loop(0, nb, _den_block, 0)
        plsc.subcore_barrier()

        @pl.when(core == 0)
        def _():
            pltpu.sync_copy(sden.at[pl.ds(tid * 640, 640)], dbuf)
            pltpu.sync_copy(dbuf, den.at[ridx, tid])

        for i in range(3):
            sl = 2 * i + core

            for i2 in range(10):
                pltpu.sync_copy(zbuf, accS.at[pl.ds(tid * 640 + i2 * 64, 64)])
            plsc.subcore_barrier()

            def _blk(q, _):
                _load_ex(q, nb, src_h, dst_h, as_h, ad_h)
                off = sl * NP
                for j in range(8):
                    for i2 in range(8):
                        idxb[j, pl.ds(16 * i2, 16)] = sblk[pl.ds(128 * j + 16 * i2, 16)] + off
                for j in range(8):
                    pltpu.async_copy(hs_h.at[idxb.at[j]], rowv, semg).wait()

                    def _scale(k, _):
                        e = plsc.load_gather(
                            exb,
                            [jnp.full((16,), j, jnp.int32), jnp.full((16,), k, jnp.int32)],
                        )
                        for j2 in range(8):
                            rowv[k, pl.ds(16 * j2, 16)] = rowv[k, pl.ds(16 * j2, 16)] * e
                        return 0

                    lax.fori_loop(0, 128, _scale, 0)
                    pltpu.sync_copy(rowv, accS.at[dstb.at[j]], add=True)
                return 0

            lax.fori_loop(0, nb, _blk, 0)
            plsc.subcore_barrier()

            for i2 in range(10):
                r0 = tid * 640 + i2 * 64
                pltpu.sync_copy(accS.at[pl.ds(r0, 64)], fbuf)
                pltpu.sync_copy(fbuf, acc_h.at[sl, pl.ds(r0, 64)])
            plsc.subcore_barrier()


@functools.lru_cache(maxsize=None)
def _edge_kernel(nbA, nbB):
    mesh = plsc.VectorSubcoreMesh(core_axis_name="c", subcore_axis_name="s")
    return pl.kernel(
        functools.partial(_edge_kernel_body, (nbA, nbB)),
        out_type=[
            jax.ShapeDtypeStruct((NSL, NP, 128), jnp.float32),
            jax.ShapeDtypeStruct((NSL, NP, 128), jnp.float32),
            jax.ShapeDtypeStruct((2, 16, 640), jnp.float32),
        ],
        mesh=mesh,
        compiler_params=pltpu.CompilerParams(needs_layout_passes=False),
        scratch_types=[
            pltpu.VMEM((1024,), jnp.int32),
            pltpu.VMEM((1024,), jnp.int32),
            pltpu.VMEM((1024,), jnp.float32),
            pltpu.VMEM((1024,), jnp.float32),
            pltpu.VMEM((8, 128), jnp.float32),
            pltpu.VMEM((8, 128), jnp.int32),
            pltpu.VMEM((8, 128), jnp.int32),
            pltpu.VMEM((16,), jnp.float32),
            pltpu.VMEM((128, 128), jnp.float32),
            pltpu.VMEM((64, 128), jnp.float32),
            pltpu.VMEM((64, 128), jnp.float32),
            pltpu.VMEM((640,), jnp.float32),
            pltpu.VMEM((640,), jnp.float32),
            pltpu.VMEM_SHARED((NP, 128), jnp.float32),
            pltpu.VMEM_SHARED((NP,), jnp.float32),
            pltpu.VMEM_SHARED((NP,), jnp.float32),
            pltpu.SemaphoreType.DMA,
        ],
    )


def _prep_edges(src, dst, nb):
    ep = 16 * nb * 1024
    e = src.shape[0]
    src = jnp.pad(src, (0, ep - e))
    dst = jnp.pad(dst, (0, ep - e), constant_values=Nd)
    return src, dst


def _slice_layout(hs):
    return hs.reshape(NP, NSL, 128).transpose(1, 0, 2).reshape(NSL * NP, 128)


def _epi_body(aA_ref, aB_ref, dA_ref, dB_ref, bA_ref, bB_ref, o_ref):
    a = aA_ref[0] / (dA_ref[...] + EPS_DEN) + bA_ref[...]
    b = aB_ref[0] / (dB_ref[...] + EPS_DEN) + bB_ref[...]
    o_ref[...] = jnp.maximum(a + b, 0.0)


def _epilogue(accA, accB, denA, denB, biasA, biasB):
    denA = denA.reshape(NP, 1)
    denB = denB.reshape(NP, 1)
    bA = biasA.reshape(1, H)
    bB = biasB.reshape(1, H)
    return pl.pallas_call(
        _epi_body,
        grid=(NP // 256, NSL),
        in_specs=[
            pl.BlockSpec((1, 256, 128), lambda i, s: (s, i, 0)),
            pl.BlockSpec((1, 256, 128), lambda i, s: (s, i, 0)),
            pl.BlockSpec((256, 1), lambda i, s: (i, 0)),
            pl.BlockSpec((256, 1), lambda i, s: (i, 0)),
            pl.BlockSpec((1, 128), lambda i, s: (0, s)),
            pl.BlockSpec((1, 128), lambda i, s: (0, s)),
        ],
        out_specs=pl.BlockSpec((256, 128), lambda i, s: (i, s)),
        out_shape=jax.ShapeDtypeStruct((NP, H), jnp.float32),
    )(accA, accB, denA, denB, bA, bB)


def _loss_body(x0_ref, x1_ref, m_ref, o_ref):
    i = pl.program_id(0)
    a = x0_ref[...] * m_ref[...]
    b = x1_ref[...] * m_ref[...]
    dot = jnp.sum(a * b, axis=1)
    na = jnp.sqrt(jnp.sum(a * a, axis=1))
    nb = jnp.sqrt(jnp.sum(b * b, axis=1))
    cos = dot / jnp.maximum(na * nb, 1e-8)
    rows = i * 256 + lax.broadcasted_iota(jnp.int32, (256,), 0)
    contrib = jnp.where(rows < Nd, 1.0 - cos, 0.0)
    s = jnp.sum(contrib).reshape(1, 1)

    @pl.when(i == 0)
    def _():
        o_ref[...] = jnp.zeros((1, 1), jnp.float32)

    o_ref[...] += s


def _cos_loss(x0, x1, mask):
    maskp = jnp.pad(mask, ((0, NP - Nd), (0, 0)))
    out = pl.pallas_call(
        _loss_body,
        grid=(NP // 256,),
        in_specs=[
            pl.BlockSpec((256, H), lambda i: (i, 0)),
            pl.BlockSpec((256, H), lambda i: (i, 0)),
            pl.BlockSpec((256, 1), lambda i: (i, 0)),
        ],
        out_specs=pl.BlockSpec((1, 1), lambda i: (0, 0)),
        out_shape=jax.ShapeDtypeStruct((1, 1), jnp.float32),
    )(x0, x1, maskp)
    return out[0, 0]


def _gather_body(x_hbm, ids_hbm, out_hbm, idxv, buf, sem):
    wid = lax.axis_index("s") * 2 + lax.axis_index("c")
    base = wid * 256
    pltpu.sync_copy(ids_hbm.at[pl.ds(base, 256)], idxv)
    for i in range(4):
        pltpu.async_copy(x_hbm.at[idxv.at[pl.ds(i * 64, 64)]], buf, sem).wait()
        pltpu.sync_copy(buf, out_hbm.at[pl.ds(base + i * 64, 64)])


@functools.lru_cache(maxsize=None)
def _gather_kernel():
    mesh = plsc.VectorSubcoreMesh(core_axis_name="c", subcore_axis_name="s")
    return pl.kernel(
        _gather_body,
        out_type=jax.ShapeDtypeStruct((8192, H), jnp.float32),
        mesh=mesh,
        compiler_params=pltpu.CompilerParams(needs_layout_passes=False),
        scratch_types=[
            pltpu.VMEM((256,), jnp.int32),
            pltpu.VMEM((64, H), jnp.float32),
            pltpu.SemaphoreType.DMA,
        ],
    )


def kernel(drug1_id, drug2_id, cell_features, x_drug, x_target, edge_dd, edge_dt, edge_rdt, edge_tt, drug_mask, target_mask, params):
    p = params
    _xd = x_drug * (1.0 - drug_mask) + drug_mask * p["mask_drug"]
    _xt = x_target * (1.0 - target_mask) + target_mask * p["mask_target"]

    loops = jnp.arange(Nd, dtype=jnp.int32)
    dd_src = jnp.concatenate([edge_dd[0], loops])
    dd_dst = jnp.concatenate([edge_dd[1], loops])
    tt_src = jnp.concatenate([edge_tt[0], loops])
    tt_dst = jnp.concatenate([edge_tt[1], loops])

    nb_dd = _rup(dd_src.shape[0], 16384) // 16384
    nb_rdt = _rup(edge_rdt.shape[1], 16384) // 16384
    nb_dt = _rup(edge_dt.shape[1], 16384) // 16384
    nb_tt = _rup(tt_src.shape[0], 16384) // 16384
    e_dd = _prep_edges(dd_src, dd_dst, nb_dd)
    e_rdt = _prep_edges(edge_rdt[0], edge_rdt[1], nb_rdt)
    e_dt = _prep_edges(edge_dt[0], edge_dt[1], nb_dt)
    e_tt = _prep_edges(tt_src, tt_dst, nb_tt)

    vd = jnp.stack(
        [
            p["dd"]["Ws"] @ p["dd"]["att_s"],
            p["dd"]["Wd"] @ p["dd"]["att_d"],
            p["dt"]["Ws"] @ p["dt"]["att_s"],
            p["rdt"]["Wd"] @ p["rdt"]["att_d"],
        ],
        axis=1,
    )
    wd_cat = jnp.concatenate([p["dd"]["Ws"], p["dt"]["Ws"], vd], axis=1)
    vt = jnp.stack(
        [
            p["rdt"]["Ws"] @ p["rdt"]["att_s"],
            p["dt"]["Wd"] @ p["dt"]["att_d"],
            p["tt"]["Ws"] @ p["tt"]["att_s"],
            p["tt"]["Wd"] @ p["tt"]["att_d"],
        ],
        axis=1,
    )
    wt_cat = jnp.concatenate([p["rdt"]["Ws"], p["tt"]["Ws"], vt], axis=1)

    outs = {}
    for li, (xd_in, xt_in) in enumerate((( _xd, _xt), (x_drug, x_target))):
        hd = _mm(jnp.pad(xd_in, ((0, NP - Nd), (0, 0))), wd_cat)
        ht = _mm(jnp.pad(xt_in, ((0, NP - Nt), (0, 0))), wt_cat)
        hs_dd = _slice_layout(hd[:, 0:H])
        hs_dt = _slice_layout(hd[:, H : 2 * H])
        as_dd, ad_dd = hd[:, 2 * H], hd[:, 2 * H + 1]
        as_dt, ad_rdt = hd[:, 2 * H + 2], hd[:, 2 * H + 3]
        hs_rdt = _slice_layout(ht[:, 0:H])
        hs_tt = _slice_layout(ht[:, H : 2 * H])
        as_rdt, ad_dt = ht[:, 2 * H], ht[:, 2 * H + 1]
        as_tt, ad_tt = ht[:, 2 * H + 2], ht[:, 2 * H + 3]

        def c16(a_s, a_d):
            A = jnp.maximum(jnp.max(a_s), 0.0) + jnp.maximum(jnp.max(a_d), 0.0)
            return jnp.full((16,), A, jnp.float32)

        accA, accB, den = _edge_kernel(nb_dd, nb_rdt)(
            e_dd[0], e_dd[1], as_dd, ad_dd, c16(as_dd, ad_dd), hs_dd,
            e_rdt[0], e_rdt[1], as_rdt, ad_rdt, c16(as_rdt, ad_rdt), hs_rdt,
        )
        outs[("d", li)] = _epilogue(accA, accB, den[0], den[1], p["dd"]["b"], p["rdt"]["b"])

        accA, accB, den = _edge_kernel(nb_dt, nb_tt)(
            e_dt[0], e_dt[1], as_dt, ad_dt, c16(as_dt, ad_dt), hs_dt,
            e_tt[0], e_tt[1], as_tt, ad_tt, c16(as_tt, ad_tt), hs_tt,
        )
        outs[("t", li)] = _epilogue(accA, accB, den[0], den[1], p["dt"]["b"], p["tt"]["b"])

    _xd_o, xd_o = outs[("d", 0)], outs[("d", 1)]
    _xt_o, xt_o = outs[("t", 0)], outs[("t", 1)]

    d_loss = _cos_loss(_xd_o, xd_o, drug_mask) / Nd
    t_loss = _cos_loss(_xt_o, xt_o, target_mask) / Nt
    loss_mae = d_loss + t_loss

    ids = jnp.concatenate([drug1_id, drug2_id]).astype(jnp.int32)
    g = _gather_kernel()(xd_o, ids)
    drug1, drug2 = g[:4096], g[4096:]

    cell = cell_features
    r0w, r0b = p["red"][0]
    cell = _mm(cell, r0w, r0b, act="relu", l2norm=True)
    r1w, r1b = p["red"][1]
    cell = _mm(cell, r1w, r1b, act="relu")
    r2w, r2b = p["red"][2]
    cell = _mm(cell, r2w, r2b, act="relu")

    hidden = jnp.concatenate([drug1, drug2, cell], axis=1)
    q0w, q0b = p["red2"][0]
    hidden = _mm(hidden, q0w, q0b, act="relu", l2norm=True)
    q1w, q1b = p["red2"][1]
    hidden = _mm(hidden, q1w, q1b, act="relu")
    q2w, q2b = p["red2"][2]
    hidden = _mm(hidden, q2w, q2b, act="relu")

    cw, cb = p["cls"]
    output = _mm(hidden, cw, cb)
    return output, loss_mae


# speedup vs baseline: 3.2412x; 1.2015x over previous
import functools

import jax
import jax.numpy as jnp
from jax import lax
from jax.experimental import pallas as pl
from jax.experimental.pallas import tpu as pltpu
from jax.experimental.pallas import tpu_sc as plsc

Nd = 10000
Nt = 10000
H = 768
NP = 10240
NSL = 6
EPS_DEN = 1e-16


def _rup(x, m):
    return (x + m - 1) // m * m


def _mm_body(x_ref, w_ref, b_ref, o_ref, *, act, l2norm):
    x = x_ref[...]
    if l2norm:
        n = jnp.sqrt(jnp.sum(x * x, axis=1, keepdims=True))
        x = x / jnp.maximum(n, 1e-12)
    y = jnp.dot(x, w_ref[...], preferred_element_type=jnp.float32)
    y = y + b_ref[...]
    if act == "relu":
        y = jnp.maximum(y, 0.0)
    o_ref[...] = y


def _mm(x, w, b=None, act=None, l2norm=False, block_m=512):
    M, K = x.shape
    _, N = w.shape
    Mp, Kp, Np = _rup(M, block_m), _rup(K, 128), _rup(N, 128)
    xp = jnp.pad(x, ((0, Mp - M), (0, Kp - K)))
    wp = jnp.pad(w, ((0, Kp - K), (0, Np - N)))
    if b is None:
        bp = jnp.zeros((1, Np), jnp.float32)
    else:
        bp = jnp.pad(b.reshape(1, N), ((0, 0), (0, Np - N)))
    out = pl.pallas_call(
        functools.partial(_mm_body, act=act, l2norm=l2norm),
        grid=(Mp // block_m,),
        in_specs=[
            pl.BlockSpec((block_m, Kp), lambda i: (i, 0)),
            pl.BlockSpec((Kp, Np), lambda i: (0, 0)),
            pl.BlockSpec((1, Np), lambda i: (0, 0)),
        ],
        out_specs=pl.BlockSpec((block_m, Np), lambda i: (i, 0)),
        out_shape=jax.ShapeDtypeStruct((Mp, Np), jnp.float32),
    )(xp, wp, bp)
    if Mp == M and Np == N:
        return out
    return out[:M, :N]


def _edge_kernel_body(
    nbs,
    srcA, dstA, asA, adA, cA, hsA,
    srcB, dstB, asB, adB, cB, hsB,
    acc, cfA, cfB,
    sblk, dblk, asg, adg, exb, cfb, dstb, idxb, c16, deng, rowv, zbuf, fbuf, zden,
    accS, sden, semg,
):
    nbA, nbB = nbs
    core = lax.axis_index("c")
    tid = lax.axis_index("s")

    def _z16(r, _):
        for j in range(8):
            zbuf[r, pl.ds(16 * j, 16)] = jnp.zeros((16,), jnp.float32)
        return 0

    lax.fori_loop(0, 16, _z16, 0)

    def _z640(i, _):
        zden[pl.ds(16 * i, 16)] = jnp.zeros((16,), jnp.float32)
        return 0

    lax.fori_loop(0, 40, _z640, 0)

    rels = (
        (nbA, srcA, dstA, asA, adA, cA, hsA, cfA),
        (nbB, srcB, dstB, asB, adB, cB, hsB, cfB),
    )

    def _load_dst(q, nb, dst_h):
        base = (tid * nb + q) * 1024
        pltpu.sync_copy(dst_h.at[pl.ds(base, 1024)], dblk)
        for j in range(8):
            for i in range(8):
                o = 128 * j + 16 * i
                dstb[j, pl.ds(16 * i, 16)] = dblk[pl.ds(o, 16)]

    for nb, src_h, dst_h, as_h, ad_h, c_h, hs_h, cf_h in rels:
        pltpu.sync_copy(c_h, c16)
        pltpu.sync_copy(zden, sden.at[pl.ds(tid * 640, 640)])
        plsc.subcore_barrier()

        def _ex_block(q, _):
            base = (tid * nb + q) * 1024
            pltpu.sync_copy(src_h.at[pl.ds(base, 1024)], sblk)
            _load_dst(q, nb, dst_h)
            for j in range(8):
                pltpu.async_copy(
                    as_h.at[sblk.at[pl.ds(128 * j, 128)]], asg.at[pl.ds(128 * j, 128)], semg
                ).wait()
                pltpu.async_copy(
                    ad_h.at[dblk.at[pl.ds(128 * j, 128)]], adg.at[pl.ds(128 * j, 128)], semg
                ).wait()
            for j in range(8):
                for i in range(8):
                    o = 128 * j + 16 * i
                    a = asg[pl.ds(o, 16)] + adg[pl.ds(o, 16)]
                    a = jnp.where(a > 0, a, 0.2 * a)
                    exb[j, pl.ds(16 * i, 16)] = jnp.exp(a - c16[...])
            for j in range(8):
                pltpu.sync_copy(exb.at[j], sden.at[dstb.at[j]], add=True)
            pltpu.sync_copy(exb, cf_h.at[core, tid * nb + q])
            return 0

        lax.fori_loop(0, nb, _ex_block, 0)
        plsc.subcore_barrier()

        def _cf_block(q, _):
            _load_dst(q, nb, dst_h)
            pltpu.sync_copy(cf_h.at[core, tid * nb + q], exb)
            for j in range(8):
                pltpu.sync_copy(sden.at[dstb.at[j]], deng)
                for i in range(8):
                    exb[j, pl.ds(16 * i, 16)] = exb[j, pl.ds(16 * i, 16)] / (
                        deng[pl.ds(16 * i, 16)] + EPS_DEN
                    )
            pltpu.sync_copy(exb, cf_h.at[core, tid * nb + q])
            return 0

        lax.fori_loop(0, nb, _cf_block, 0)
        plsc.subcore_barrier()

    for i in range(3):
        sl = 2 * i + core

        for i2 in range(40):
            pltpu.sync_copy(zbuf, accS.at[pl.ds(tid * 640 + i2 * 16, 16)])
        plsc.subcore_barrier()

        for nb, src_h, dst_h, as_h, ad_h, c_h, hs_h, cf_h in rels:

            def _blk(q, _):
                base = (tid * nb + q) * 1024
                pltpu.sync_copy(src_h.at[pl.ds(base, 1024)], sblk)
                _load_dst(q, nb, dst_h)
                pltpu.sync_copy(cf_h.at[core, tid * nb + q], cfb)
                off = sl * NP
                for j in range(8):
                    for i2 in range(8):
                        idxb[j, pl.ds(16 * i2, 16)] = sblk[pl.ds(128 * j + 16 * i2, 16)] + off
                for j in range(8):
                    pltpu.async_copy(hs_h.at[idxb.at[j]], rowv, semg).wait()

                    def _scale(k, _):
                        e = plsc.load_gather(
                            cfb,
                            [jnp.full((16,), j, jnp.int32), jnp.full((16,), k, jnp.int32)],
                        )
                        for j2 in range(8):
                            rowv[k, pl.ds(16 * j2, 16)] = rowv[k, pl.ds(16 * j2, 16)] * e
                        return 0

                    lax.fori_loop(0, 128, _scale, 0)
                    pltpu.sync_copy(rowv, accS.at[dstb.at[j]], add=True)
                return 0

            lax.fori_loop(0, nb, _blk, 0)
        plsc.subcore_barrier()

        for i2 in range(40):
            r0 = tid * 640 + i2 * 16
            pltpu.sync_copy(accS.at[pl.ds(r0, 16)], fbuf)
            pltpu.sync_copy(fbuf, acc.at[sl, pl.ds(r0, 16)])
        plsc.subcore_barrier()


@functools.lru_cache(maxsize=None)
def _edge_kernel(nbA, nbB):
    mesh = plsc.VectorSubcoreMesh(core_axis_name="c", subcore_axis_name="s")
    return pl.kernel(
        functools.partial(_edge_kernel_body, (nbA, nbB)),
        out_type=[
            jax.ShapeDtypeStruct((NSL, NP, 128), jnp.float32),
            jax.ShapeDtypeStruct((2, 16 * nbA, 8, 128), jnp.float32),
            jax.ShapeDtypeStruct((2, 16 * nbB, 8, 128), jnp.float32),
        ],
        mesh=mesh,
        compiler_params=pltpu.CompilerParams(needs_layout_passes=False),
        scratch_types=[
            pltpu.VMEM((1024,), jnp.int32),
            pltpu.VMEM((1024,), jnp.int32),
            pltpu.VMEM((1024,), jnp.float32),
            pltpu.VMEM((1024,), jnp.float32),
            pltpu.VMEM((8, 128), jnp.float32),
            pltpu.VMEM((8, 128), jnp.float32),
            pltpu.VMEM((8, 128), jnp.int32),
            pltpu.VMEM((8, 128), jnp.int32),
            pltpu.VMEM((16,), jnp.float32),
            pltpu.VMEM((128,), jnp.float32),
            pltpu.VMEM((128, 128), jnp.float32),
            pltpu.VMEM((16, 128), jnp.float32),
            pltpu.VMEM((16, 128), jnp.float32),
            pltpu.VMEM((640,), jnp.float32),
            pltpu.VMEM_SHARED((NP, 128), jnp.float32),
            pltpu.VMEM_SHARED((NP,), jnp.float32),
            pltpu.SemaphoreType.DMA,
        ],
    )


def _prep_edges(src, dst, nb):
    ep = 16 * nb * 1024
    e = src.shape[0]
    src = jnp.pad(src, (0, ep - e))
    dst = jnp.pad(dst, (0, ep - e), constant_values=Nd)
    return src, dst


def _slice_layout(hs):
    return hs.reshape(NP, NSL, 128).transpose(1, 0, 2).reshape(NSL * NP, 128)


def _epi_body(a_ref, b_ref, o_ref):
    o_ref[...] = jnp.maximum(a_ref[0] + b_ref[...], 0.0)


def _epilogue(acc, biasA, biasB):
    b = (biasA + biasB).reshape(1, H)
    return pl.pallas_call(
        _epi_body,
        grid=(NP // 256, NSL),
        in_specs=[
            pl.BlockSpec((1, 256, 128), lambda i, s: (s, i, 0)),
            pl.BlockSpec((1, 128), lambda i, s: (0, s)),
        ],
        out_specs=pl.BlockSpec((256, 128), lambda i, s: (i, s)),
        out_shape=jax.ShapeDtypeStruct((NP, H), jnp.float32),
    )(acc, b)


def _loss_body(x0_ref, x1_ref, m_ref, o_ref):
    i = pl.program_id(0)
    a = x0_ref[...] * m_ref[...]
    b = x1_ref[...] * m_ref[...]
    dot = jnp.sum(a * b, axis=1)
    na = jnp.sqrt(jnp.sum(a * a, axis=1))
    nb = jnp.sqrt(jnp.sum(b * b, axis=1))
    cos = dot / jnp.maximum(na * nb, 1e-8)
    rows = i * 256 + lax.broadcasted_iota(jnp.int32, (256,), 0)
    contrib = jnp.where(rows < Nd, 1.0 - cos, 0.0)
    s = jnp.sum(contrib).reshape(1, 1)

    @pl.when(i == 0)
    def _():
        o_ref[...] = jnp.zeros((1, 1), jnp.float32)

    o_ref[...] += s


def _cos_loss(x0, x1, mask):
    maskp = jnp.pad(mask, ((0, NP - Nd), (0, 0)))
    out = pl.pallas_call(
        _loss_body,
        grid=(NP // 256,),
        in_specs=[
            pl.BlockSpec((256, H), lambda i: (i, 0)),
            pl.BlockSpec((256, H), lambda i: (i, 0)),
            pl.BlockSpec((256, 1), lambda i: (i, 0)),
        ],
        out_specs=pl.BlockSpec((1, 1), lambda i: (0, 0)),
        out_shape=jax.ShapeDtypeStruct((1, 1), jnp.float32),
    )(x0, x1, maskp)
    return out[0, 0]


def _gather_body(x_hbm, ids_hbm, out_hbm, idxv, buf, sem):
    wid = lax.axis_index("s") * 2 + lax.axis_index("c")
    base = wid * 256
    pltpu.sync_copy(ids_hbm.at[pl.ds(base, 256)], idxv)
    for i in range(4):
        pltpu.async_copy(x_hbm.at[idxv.at[pl.ds(i * 64, 64)]], buf, sem).wait()
        pltpu.sync_copy(buf, out_hbm.at[pl.ds(base + i * 64, 64)])


@functools.lru_cache(maxsize=None)
def _gather_kernel():
    mesh = plsc.VectorSubcoreMesh(core_axis_name="c", subcore_axis_name="s")
    return pl.kernel(
        _gather_body,
        out_type=jax.ShapeDtypeStruct((8192, H), jnp.float32),
        mesh=mesh,
        compiler_params=pltpu.CompilerParams(needs_layout_passes=False),
        scratch_types=[
            pltpu.VMEM((256,), jnp.int32),
            pltpu.VMEM((64, H), jnp.float32),
            pltpu.SemaphoreType.DMA,
        ],
    )


def kernel(drug1_id, drug2_id, cell_features, x_drug, x_target, edge_dd, edge_dt, edge_rdt, edge_tt, drug_mask, target_mask, params):
    p = params
    _xd = x_drug * (1.0 - drug_mask) + drug_mask * p["mask_drug"]
    _xt = x_target * (1.0 - target_mask) + target_mask * p["mask_target"]

    loops = jnp.arange(Nd, dtype=jnp.int32)
    dd_src = jnp.concatenate([edge_dd[0], loops])
    dd_dst = jnp.concatenate([edge_dd[1], loops])
    tt_src = jnp.concatenate([edge_tt[0], loops])
    tt_dst = jnp.concatenate([edge_tt[1], loops])

    nb_dd = _rup(dd_src.shape[0], 16384) // 16384
    nb_rdt = _rup(edge_rdt.shape[1], 16384) // 16384
    nb_dt = _rup(edge_dt.shape[1], 16384) // 16384
    nb_tt = _rup(tt_src.shape[0], 16384) // 16384
    e_dd = _prep_edges(dd_src, dd_dst, nb_dd)
    e_rdt = _prep_edges(edge_rdt[0], edge_rdt[1], nb_rdt)
    e_dt = _prep_edges(edge_dt[0], edge_dt[1], nb_dt)
    e_tt = _prep_edges(tt_src, tt_dst, nb_tt)

    vd = jnp.stack(
        [
            p["dd"]["Ws"] @ p["dd"]["att_s"],
            p["dd"]["Wd"] @ p["dd"]["att_d"],
            p["dt"]["Ws"] @ p["dt"]["att_s"],
            p["rdt"]["Wd"] @ p["rdt"]["att_d"],
        ],
        axis=1,
    )
    wd_cat = jnp.concatenate([p["dd"]["Ws"], p["dt"]["Ws"], vd], axis=1)
    vt = jnp.stack(
        [
            p["rdt"]["Ws"] @ p["rdt"]["att_s"],
            p["dt"]["Wd"] @ p["dt"]["att_d"],
            p["tt"]["Ws"] @ p["tt"]["att_s"],
            p["tt"]["Wd"] @ p["tt"]["att_d"],
        ],
        axis=1,
    )
    wt_cat = jnp.concatenate([p["rdt"]["Ws"], p["tt"]["Ws"], vt], axis=1)

    outs = {}
    for li, (xd_in, xt_in) in enumerate((( _xd, _xt), (x_drug, x_target))):
        hd = _mm(jnp.pad(xd_in, ((0, NP - Nd), (0, 0))), wd_cat)
        ht = _mm(jnp.pad(xt_in, ((0, NP - Nt), (0, 0))), wt_cat)
        hs_dd = _slice_layout(hd[:, 0:H])
        hs_dt = _slice_layout(hd[:, H : 2 * H])
        as_dd, ad_dd = hd[:, 2 * H], hd[:, 2 * H + 1]
        as_dt, ad_rdt = hd[:, 2 * H + 2], hd[:, 2 * H + 3]
        hs_rdt = _slice_layout(ht[:, 0:H])
        hs_tt = _slice_layout(ht[:, H : 2 * H])
        as_rdt, ad_dt = ht[:, 2 * H], ht[:, 2 * H + 1]
        as_tt, ad_tt = ht[:, 2 * H + 2], ht[:, 2 * H + 3]

        def c16(a_s, a_d):
            A = jnp.maximum(jnp.max(a_s), 0.0) + jnp.maximum(jnp.max(a_d), 0.0)
            return jnp.full((16,), A, jnp.float32)

        acc_d, _, _ = _edge_kernel(nb_dd, nb_rdt)(
            e_dd[0], e_dd[1], as_dd, ad_dd, c16(as_dd, ad_dd), hs_dd,
            e_rdt[0], e_rdt[1], as_rdt, ad_rdt, c16(as_rdt, ad_rdt), hs_rdt,
        )
        outs[("d", li)] = _epilogue(acc_d, p["dd"]["b"], p["rdt"]["b"])

        acc_t, _, _ = _edge_kernel(nb_dt, nb_tt)(
            e_dt[0], e_dt[1], as_dt, ad_dt, c16(as_dt, ad_dt), hs_dt,
            e_tt[0], e_tt[1], as_tt, ad_tt, c16(as_tt, ad_tt), hs_tt,
        )
        outs[("t", li)] = _epilogue(acc_t, p["dt"]["b"], p["tt"]["b"])

    _xd_o, xd_o = outs[("d", 0)], outs[("d", 1)]
    _xt_o, xt_o = outs[("t", 0)], outs[("t", 1)]

    d_loss = _cos_loss(_xd_o, xd_o, drug_mask) / Nd
    t_loss = _cos_loss(_xt_o, xt_o, target_mask) / Nt
    loss_mae = d_loss + t_loss

    ids = jnp.concatenate([drug1_id, drug2_id]).astype(jnp.int32)
    g = _gather_kernel()(xd_o, ids)
    drug1, drug2 = g[:4096], g[4096:]

    cell = cell_features
    r0w, r0b = p["red"][0]
    cell = _mm(cell, r0w, r0b, act="relu", l2norm=True)
    r1w, r1b = p["red"][1]
    cell = _mm(cell, r1w, r1b, act="relu")
    r2w, r2b = p["red"][2]
    cell = _mm(cell, r2w, r2b, act="relu")

    hidden = jnp.concatenate([drug1, drug2, cell], axis=1)
    q0w, q0b = p["red2"][0]
    hidden = _mm(hidden, q0w, q0b, act="relu", l2norm=True)
    q1w, q1b = p["red2"][1]
    hidden = _mm(hidden, q1w, q1b, act="relu")
    q2w, q2b = p["red2"][2]
    hidden = _mm(hidden, q2w, q2b, act="relu")

    cw, cb = p["cls"]
    output = _mm(hidden, cw, cb)
    return output, loss_mae
